# baseline (speedup 1.0000x reference)
import jax
import jax.numpy as jnp
from jax.experimental import pallas as pl

HEADS0 = 8


def _gat_layer(x, row, col, Wq, bq, Wk, bk, Wv, b, num_heads, activation):
    n = x.shape[0]
    e = row.shape[0]
    Q = jax.nn.relu(x @ Wq + bq)[row]
    K = jax.nn.relu(x @ Wk + bk)[col]
    V = x @ Wv
    att_units = Q.shape[-1]
    Qh = Q.reshape(e, num_heads, att_units // num_heads)
    Kh = K.reshape(e, num_heads, att_units // num_heads)
    att = jnp.sum(Qh * Kh, axis=-1)
    att_max = jax.ops.segment_max(att, row, num_segments=n)
    att_exp = jnp.exp(att - att_max[row])
    denom = jax.ops.segment_sum(att_exp, row, num_segments=n)
    alpha = att_exp / (denom[row] + 1e-16)
    Vh = V[col].reshape(e, num_heads, V.shape[-1] // num_heads)
    out = jax.ops.segment_sum(alpha[:, :, None] * Vh, row, num_segments=n)
    out = out.reshape(n, -1) + b
    if activation is not None:
        out = activation(out)
    return out


def _addb(h_ref, b_ref, o_ref):
    o_ref[...] = h_ref[...] + b_ref[...]


def kernel(x, edge_index, Wq0, bq0, Wk0, bk0, Wv0, b0, Wq1, bq1, Wk1, bk1, Wv1, b1):
    n = x.shape[0]
    loop = jnp.arange(n, dtype=edge_index.dtype)
    row = jnp.concatenate([edge_index[0], loop])
    col = jnp.concatenate([edge_index[1], loop])
    h = _gat_layer(x, row, col, Wq0, bq0, Wk0, bk0, Wv0, b0, HEADS0, jax.nn.relu)
    h = _gat_layer(h, row, col, Wq1, bq1, Wk1, bk1, Wv1, 0.0, 1, None)
    b1p = jnp.broadcast_to(b1, h.shape)
    out = pl.pallas_call(
        _addb,
        out_shape=jax.ShapeDtypeStruct(h.shape, h.dtype),
    )(h, b1p)
    return out


# speedup vs baseline: 23.0888x; 23.0888x over previous
import dataclasses
import functools

import jax
import jax.numpy as jnp
from jax import lax
from jax.experimental import pallas as pl
from jax.experimental.pallas import tpu as pltpu
from jax.experimental.pallas import tpu_sc as plsc

N = 10000
E = 320000
NP = 10240
ROWS_PER_TILE = NP // 16
EC_BLOCKS = 80
EB = 128
EP = 32 * EC_BLOCKS * EB
DH = 8
TCB = 512
F32 = jnp.float32
HI = lax.Precision.HIGHEST


def _exp_precise(x):
    y = x * 1.4426950408889634
    yc = jnp.clip(y, -125.0, 125.0)
    n = (yc + jnp.where(yc >= 0.0, 0.5, -0.5)).astype(jnp.int32)
    t = (yc - n.astype(F32)) * 0.6931471805599453
    p = 1.0 + t * (1.0 + t * (0.5 + t * (0.16666667 + t * (0.041666668
        + t * 0.008333334))))
    return p * lax.bitcast_convert_type((n + 127) << 23, F32)


def _tc_a_body(x_ref, w_ref, b_ref, t0_ref, ii_ref):
    y = (jnp.dot(x_ref[...], w_ref[...], preferred_element_type=F32,
                 precision=HI) + b_ref[...])
    q = jax.nn.relu(y[:, 0:8])
    k = jax.nn.relu(y[:, 8:16])
    v = y[:, 16:80]
    zp = jnp.zeros((y.shape[0], 32), F32)
    t0_ref[...] = jnp.concatenate([q, q, k, k, v, zp], axis=1)
    d8 = 0.5 * _exp_precise(q * k)
    rep = (lax.broadcasted_iota(jnp.int32, (8, 64), 1) // DH
           == lax.broadcasted_iota(jnp.int32, (8, 64), 0)).astype(F32)
    ai = jnp.dot(d8, rep, preferred_element_type=F32, precision=HI) * v
    z16 = jnp.zeros((y.shape[0], 16), F32)
    ii_ref[...] = jnp.concatenate([d8, d8, z16, ai, zp], axis=1)


def _tc_b_body(ap_ref, b0_ref, wq_ref, bq_ref, wk_ref, bk_ref,
               wv_ref, t1_ref, i1_ref):
    d = ap_ref[0, :, 0:8] + ap_ref[1, :, 0:8]
    a = ap_ref[0, :, 32:96] + ap_ref[1, :, 32:96]
    inv = 1.0 / (d + 1e-16)
    rep = (lax.broadcasted_iota(jnp.int32, (8, 64), 1) // DH
           == lax.broadcasted_iota(jnp.int32, (8, 64), 0)).astype(F32)
    h0 = jax.nn.relu(
        a * jnp.dot(inv, rep, preferred_element_type=F32, precision=HI)
        + b0_ref[...])
    q1 = jax.nn.relu(jnp.dot(h0, wq_ref[...], preferred_element_type=F32,
                             precision=HI) + bq_ref[...])
    k1 = jax.nn.relu(jnp.dot(h0, wk_ref[...], preferred_element_type=F32,
                             precision=HI) + bk_ref[...])
    v1 = jnp.dot(h0, wv_ref[...], preferred_element_type=F32, precision=HI)
    rows = q1.shape[0]
    lanes16 = jnp.ones((1, 16), F32)
    ones = jnp.ones((rows, 1), F32)
    z8 = jnp.zeros((rows, 8), F32)
    z80 = jnp.zeros((rows, 80), F32)
    w1 = jnp.concatenate([ones, v1, z8], axis=1)
    t1_ref[...] = jnp.concatenate([q1 * lanes16, k1 * lanes16, w1, z80],
                                  axis=1)
    z32 = jnp.zeros((rows, 32), F32)
    i1_ref[...] = jnp.concatenate(
        [z32, (0.5 * _exp_precise(q1 * k1)) * w1, z80], axis=1)


def _tc_c_body(ap_ref, b1_ref, o_ref):
    s = ap_ref[0] + ap_ref[1]
    o_ref[...] = s[:, 33:40] / (s[:, 32:33] + 1e-16) + b1_ref[...]


def _take16(vec, pat):
    dn = lax.GatherDimensionNumbers(offset_dims=(), collapsed_slice_dims=(0,),
                                    start_index_map=(0,))
    return lax.gather(vec, pat[:, None], dn, slice_sizes=(1,),
                      mode=lax.GatherScatterMode.PROMISE_IN_BOUNDS)


def _sc_edge_pass(row_hbm, col_hbm, tab_hbm, init_hbm, aout_hbm,
                  rowt, colt, gb, qs, acc):
    c = lax.axis_index("c")
    s = lax.axis_index("s")
    wid = c * 16 + s
    pltpu.sync_copy(row_hbm.at[wid], rowt)
    pltpu.sync_copy(col_hbm.at[wid], colt)
    rows = pl.ds(s * ROWS_PER_TILE, ROWS_PER_TILE)
    pltpu.sync_copy(init_hbm.at[rows], acc.at[rows])
    plsc.subcore_barrier()

    lane = lax.iota(jnp.int32, 16)

    @pl.loop(0, EC_BLOCKS)
    def _blk(b):
        ir = rowt.at[b]
        ic = colt.at[b]
        pltpu.sync_copy(tab_hbm.at[ir], gb)
        for e in range(EB):
            qs[pl.ds(16 * e, 16)] = gb.at[e][pl.ds(0, 16)]
        pltpu.sync_copy(tab_hbm.at[ic], gb)
        for e in range(EB):
            ex = _exp_precise(qs[pl.ds(16 * e, 16)] * gb.at[e][pl.ds(16, 16)])
            gb.at[e][pl.ds(0, 16)] = ex
            for r in range(4):
                pat = (lane >= 8).astype(jnp.int32) + 2 * r
                exf = _take16(ex, pat)
                sl = pl.ds(32 + 16 * r, 16)
                gb.at[e][sl] = exf * gb.at[e][sl]
        pltpu.sync_copy(gb, acc.at[ir], add=True)

    plsc.subcore_barrier()
    pltpu.sync_copy(acc.at[rows], aout_hbm.at[c, rows])


def kernel(x, edge_index, Wq0, bq0, Wk0, bk0, Wv0, b0, Wq1, bq1, Wk1, bk1,
           Wv1, b1):
    xp = jnp.pad(x, ((0, NP - N), (0, 0)))
    ei = edge_index.astype(jnp.int32)
    pad = jnp.full((EP - E,), N, jnp.int32)
    rowp = jnp.concatenate([ei[0], pad])
    colp = jnp.concatenate([ei[1], pad])
    row2d = rowp.reshape(32, EC_BLOCKS, EB)
    col2d = colp.reshape(32, EC_BLOCKS, EB)

    w0 = jnp.concatenate([Wq0, Wk0, Wv0], axis=1)
    b0cat = jnp.concatenate([bq0, bk0, jnp.zeros((64,), F32)]).reshape(1, 80)

    grid = NP // TCB
    t0, ii = pl.pallas_call(
        _tc_a_body,
        grid=(grid,),
        in_specs=[
            pl.BlockSpec((TCB, 128), lambda i: (i, 0)),
            pl.BlockSpec((128, 80), lambda i: (0, 0)),
            pl.BlockSpec((1, 80), lambda i: (0, 0)),
        ],
        out_specs=[
            pl.BlockSpec((TCB, 128), lambda i: (i, 0)),
            pl.BlockSpec((TCB, 128), lambda i: (i, 0)),
        ],
        out_shape=[
            jax.ShapeDtypeStruct((NP, 128), F32),
            jax.ShapeDtypeStruct((NP, 128), F32),
        ],
    )(xp, w0, b0cat)

    mesh = plsc.VectorSubcoreMesh(core_axis_name="c", subcore_axis_name="s")
    cp = pltpu.CompilerParams()
    if "needs_layout_passes" in pltpu.CompilerParams.__dataclass_fields__:
        cp = dataclasses.replace(cp, needs_layout_passes=False)
    edge_pass = functools.partial(
        pl.kernel,
        compiler_params=cp,
        out_type=jax.ShapeDtypeStruct((2, NP, 128), F32),
        mesh=mesh,
        scratch_types=[
            pltpu.VMEM((EC_BLOCKS, EB), jnp.int32),
            pltpu.VMEM((EC_BLOCKS, EB), jnp.int32),
            pltpu.VMEM((EB, 128), F32),
            pltpu.VMEM((EB * 16,), F32),
            pltpu.VMEM_SHARED((NP, 128), F32),
        ],
    )(_sc_edge_pass)

    aparts = edge_pass(row2d, col2d, t0, ii)

    t1, i1 = pl.pallas_call(
        _tc_b_body,
        grid=(grid,),
        in_specs=[
            pl.BlockSpec((2, TCB, 128), lambda i: (0, i, 0)),
            pl.BlockSpec((1, 64), lambda i: (0, 0)),
            pl.BlockSpec((64, 1), lambda i: (0, 0)),
            pl.BlockSpec((1, 1), lambda i: (0, 0)),
            pl.BlockSpec((64, 1), lambda i: (0, 0)),
            pl.BlockSpec((1, 1), lambda i: (0, 0)),
            pl.BlockSpec((64, 7), lambda i: (0, 0)),
        ],
        out_specs=[
            pl.BlockSpec((TCB, 128), lambda i: (i, 0)),
            pl.BlockSpec((TCB, 128), lambda i: (i, 0)),
        ],
        out_shape=[
            jax.ShapeDtypeStruct((NP, 128), F32),
            jax.ShapeDtypeStruct((NP, 128), F32),
        ],
    )(aparts, b0.reshape(1, 64), Wq1, bq1.reshape(1, 1), Wk1,
      bk1.reshape(1, 1), Wv1)

    a1parts = edge_pass(row2d, col2d, t1, i1)

    out = pl.pallas_call(
        _tc_c_body,
        grid=(grid,),
        in_specs=[
            pl.BlockSpec((2, TCB, 128), lambda i: (0, i, 0)),
            pl.BlockSpec((1, 7), lambda i: (0, 0)),
        ],
        out_specs=pl.BlockSpec((TCB, 7), lambda i: (i, 0)),
        out_shape=jax.ShapeDtypeStruct((NP, 7), F32),
    )(a1parts, b1.reshape(1, 7))

    return out[:N]


# speedup vs baseline: 23.9546x; 1.0375x over previous
import dataclasses
import functools

import jax
import jax.numpy as jnp
from jax import lax
from jax.experimental import pallas as pl
from jax.experimental.pallas import tpu as pltpu
from jax.experimental.pallas import tpu_sc as plsc

N = 10000
E = 320000
NP = 10240
ROWS_PER_TILE = NP // 16
EC_BLOCKS = 80
EB = 128
EP = 32 * EC_BLOCKS * EB
DH = 8
TCB = 512
F32 = jnp.float32
HI = lax.Precision.HIGHEST


def _exp_precise(x):
    y = x * 1.4426950408889634
    yc = jnp.clip(y, -125.0, 125.0)
    n = (yc + jnp.where(yc >= 0.0, 0.5, -0.5)).astype(jnp.int32)
    t = (yc - n.astype(F32)) * 0.6931471805599453
    p = 1.0 + t * (1.0 + t * (0.5 + t * (0.16666667 + t * (0.041666668
        + t * 0.008333334))))
    return p * lax.bitcast_convert_type((n + 127) << 23, F32)


def _tc_a_body(x_ref, w_ref, b_ref, t0_ref, ii_ref):
    y = (jnp.dot(x_ref[...], w_ref[...], preferred_element_type=F32,
                 precision=HI) + b_ref[...])
    q = jax.nn.relu(y[:, 0:8])
    k = jax.nn.relu(y[:, 8:16])
    v = y[:, 16:80]
    zp = jnp.zeros((y.shape[0], 32), F32)
    t0_ref[...] = jnp.concatenate([q, q, k, k, v, zp], axis=1)
    d8 = 0.5 * _exp_precise(q * k)
    rep = (lax.broadcasted_iota(jnp.int32, (8, 64), 1) // DH
           == lax.broadcasted_iota(jnp.int32, (8, 64), 0)).astype(F32)
    ai = jnp.dot(d8, rep, preferred_element_type=F32, precision=HI) * v
    z16 = jnp.zeros((y.shape[0], 16), F32)
    ii_ref[...] = jnp.concatenate([d8, d8, z16, ai, zp], axis=1)


def _tc_b_body(ap_ref, b0_ref, wq_ref, bq_ref, wk_ref, bk_ref,
               wv_ref, t1_ref, i1_ref):
    d = ap_ref[0, :, 0:8] + ap_ref[1, :, 0:8]
    a = ap_ref[0, :, 32:96] + ap_ref[1, :, 32:96]
    inv = 1.0 / (d + 1e-16)
    rep = (lax.broadcasted_iota(jnp.int32, (8, 64), 1) // DH
           == lax.broadcasted_iota(jnp.int32, (8, 64), 0)).astype(F32)
    h0 = jax.nn.relu(
        a * jnp.dot(inv, rep, preferred_element_type=F32, precision=HI)
        + b0_ref[...])
    q1 = jax.nn.relu(jnp.dot(h0, wq_ref[...], preferred_element_type=F32,
                             precision=HI) + bq_ref[...])
    k1 = jax.nn.relu(jnp.dot(h0, wk_ref[...], preferred_element_type=F32,
                             precision=HI) + bk_ref[...])
    v1 = jnp.dot(h0, wv_ref[...], preferred_element_type=F32, precision=HI)
    rows = q1.shape[0]
    lanes16 = jnp.ones((1, 16), F32)
    ones = jnp.ones((rows, 1), F32)
    z8 = jnp.zeros((rows, 8), F32)
    z80 = jnp.zeros((rows, 80), F32)
    w1 = jnp.concatenate([ones, v1, z8], axis=1)
    t1_ref[...] = jnp.concatenate([q1 * lanes16, k1 * lanes16, w1, z80],
                                  axis=1)
    z32 = jnp.zeros((rows, 32), F32)
    i1_ref[...] = jnp.concatenate(
        [z32, (0.5 * _exp_precise(q1 * k1)) * w1, z80], axis=1)


def _tc_c_body(ap_ref, b1_ref, o_ref):
    s = ap_ref[0] + ap_ref[1]
    o_ref[...] = s[:, 33:40] / (s[:, 32:33] + 1e-16) + b1_ref[...]


def _take16(vec, pat):
    dn = lax.GatherDimensionNumbers(offset_dims=(), collapsed_slice_dims=(0,),
                                    start_index_map=(0,))
    return lax.gather(vec, pat[:, None], dn, slice_sizes=(1,),
                      mode=lax.GatherScatterMode.PROMISE_IN_BOUNDS)


def _sc_edge_pass(row_hbm, col_hbm, tab_hbm, init_hbm, aout_hbm,
                  rowt, colt, ga, gc, acc, sem_r, sem_c):
    c = lax.axis_index("c")
    s = lax.axis_index("s")
    wid = c * 16 + s
    rows = pl.ds(s * ROWS_PER_TILE, ROWS_PER_TILE)
    pltpu.sync_copy(init_hbm.at[rows], acc.at[rows])
    plsc.subcore_barrier()

    lane = lax.iota(jnp.int32, 16)
    pblk = EC_BLOCKS // 5

    @pl.loop(0, 5)
    def _phase(ph):
        off = pl.multiple_of(ph * pblk, 8)
        pltpu.sync_copy(row_hbm.at[wid, pl.ds(off, pblk)], rowt)
        pltpu.sync_copy(col_hbm.at[wid, pl.ds(off, pblk)], colt)

        @pl.loop(0, pblk)
        def _blk(b):
            ir = rowt.at[b]
            ic = colt.at[b]
            cp_r = pltpu.async_copy(tab_hbm.at[ir], ga, sem_r)
            cp_c = pltpu.async_copy(tab_hbm.at[ic], gc, sem_c)
            cp_r.wait()
            cp_c.wait()
            for e in range(EB):
                ex = _exp_precise(ga.at[e][pl.ds(0, 16)]
                                  * gc.at[e][pl.ds(16, 16)])
                gc.at[e][pl.ds(0, 16)] = ex
                for r in range(4):
                    pat = (lane >= 8).astype(jnp.int32) + 2 * r
                    exf = _take16(ex, pat)
                    sl = pl.ds(32 + 16 * r, 16)
                    gc.at[e][sl] = exf * gc.at[e][sl]
            pltpu.sync_copy(gc, acc.at[ir], add=True)

    plsc.subcore_barrier()
    pltpu.sync_copy(acc.at[rows], aout_hbm.at[c, rows])


def kernel(x, edge_index, Wq0, bq0, Wk0, bk0, Wv0, b0, Wq1, bq1, Wk1, bk1,
           Wv1, b1):
    xp = jnp.pad(x, ((0, NP - N), (0, 0)))
    ei = edge_index.astype(jnp.int32)
    pad = jnp.full((EP - E,), N, jnp.int32)
    rowp = jnp.concatenate([ei[0], pad])
    colp = jnp.concatenate([ei[1], pad])
    row2d = rowp.reshape(32, EC_BLOCKS, EB)
    col2d = colp.reshape(32, EC_BLOCKS, EB)

    w0 = jnp.concatenate([Wq0, Wk0, Wv0], axis=1)
    b0cat = jnp.concatenate([bq0, bk0, jnp.zeros((64,), F32)]).reshape(1, 80)

    grid = NP // TCB
    t0, ii = pl.pallas_call(
        _tc_a_body,
        grid=(grid,),
        in_specs=[
            pl.BlockSpec((TCB, 128), lambda i: (i, 0)),
            pl.BlockSpec((128, 80), lambda i: (0, 0)),
            pl.BlockSpec((1, 80), lambda i: (0, 0)),
        ],
        out_specs=[
            pl.BlockSpec((TCB, 128), lambda i: (i, 0)),
            pl.BlockSpec((TCB, 128), lambda i: (i, 0)),
        ],
        out_shape=[
            jax.ShapeDtypeStruct((NP, 128), F32),
            jax.ShapeDtypeStruct((NP, 128), F32),
        ],
    )(xp, w0, b0cat)

    mesh = plsc.VectorSubcoreMesh(core_axis_name="c", subcore_axis_name="s")
    cp = pltpu.CompilerParams()
    if "needs_layout_passes" in pltpu.CompilerParams.__dataclass_fields__:
        cp = dataclasses.replace(cp, needs_layout_passes=False)
    edge_pass = functools.partial(
        pl.kernel,
        compiler_params=cp,
        out_type=jax.ShapeDtypeStruct((2, NP, 128), F32),
        mesh=mesh,
        scratch_types=[
            pltpu.VMEM((EC_BLOCKS // 5, EB), jnp.int32),
            pltpu.VMEM((EC_BLOCKS // 5, EB), jnp.int32),
            pltpu.VMEM((EB, 128), F32),
            pltpu.VMEM((EB, 128), F32),
            pltpu.VMEM_SHARED((NP, 128), F32),
            pltpu.SemaphoreType.DMA,
            pltpu.SemaphoreType.DMA,
        ],
    )(_sc_edge_pass)

    aparts = edge_pass(row2d, col2d, t0, ii)

    t1, i1 = pl.pallas_call(
        _tc_b_body,
        grid=(grid,),
        in_specs=[
            pl.BlockSpec((2, TCB, 128), lambda i: (0, i, 0)),
            pl.BlockSpec((1, 64), lambda i: (0, 0)),
            pl.BlockSpec((64, 1), lambda i: (0, 0)),
            pl.BlockSpec((1, 1), lambda i: (0, 0)),
            pl.BlockSpec((64, 1), lambda i: (0, 0)),
            pl.BlockSpec((1, 1), lambda i: (0, 0)),
            pl.BlockSpec((64, 7), lambda i: (0, 0)),
        ],
        out_specs=[
            pl.BlockSpec((TCB, 128), lambda i: (i, 0)),
            pl.BlockSpec((TCB, 128), lambda i: (i, 0)),
        ],
        out_shape=[
            jax.ShapeDtypeStruct((NP, 128), F32),
            jax.ShapeDtypeStruct((NP, 128), F32),
        ],
    )(aparts, b0.reshape(1, 64), Wq1, bq1.reshape(1, 1), Wk1,
      bk1.reshape(1, 1), Wv1)

    a1parts = edge_pass(row2d, col2d, t1, i1)

    out = pl.pallas_call(
        _tc_c_body,
        grid=(grid,),
        in_specs=[
            pl.BlockSpec((2, TCB, 128), lambda i: (0, i, 0)),
            pl.BlockSpec((1, 7), lambda i: (0, 0)),
        ],
        out_specs=pl.BlockSpec((TCB, 7), lambda i: (i, 0)),
        out_shape=jax.ShapeDtypeStruct((NP, 7), F32),
    )(a1parts, b1.reshape(1, 7))

    return out[:N]


# speedup vs baseline: 27.8570x; 1.1629x over previous
import dataclasses
import functools

import jax
import jax.numpy as jnp
from jax import lax
from jax.experimental import pallas as pl
from jax.experimental.pallas import tpu as pltpu
from jax.experimental.pallas import tpu_sc as plsc

N = 10000
E = 320000
NP = 10240
ROWS_PER_TILE = NP // 16
EC_BLOCKS = 80
EB = 128
EP = 32 * EC_BLOCKS * EB
DH = 8
TCB = 512
F32 = jnp.float32
HI = lax.Precision.HIGHEST


def _exp_precise(x):
    y = x * 1.4426950408889634
    yc = jnp.clip(y, -125.0, 125.0)
    n = (yc + jnp.where(yc >= 0.0, 0.5, -0.5)).astype(jnp.int32)
    t = (yc - n.astype(F32)) * 0.6931471805599453
    p = 1.0 + t * (1.0 + t * (0.5 + t * (0.16666667 + t * (0.041666668
        + t * 0.008333334))))
    return p * lax.bitcast_convert_type((n + 127) << 23, F32)


def _tc_a_body(x_ref, w_ref, b_ref, t0_ref, ii_ref):
    y = (jnp.dot(x_ref[...], w_ref[...], preferred_element_type=F32,
                 precision=HI) + b_ref[...])
    q = jax.nn.relu(y[:, 0:8])
    k = jax.nn.relu(y[:, 8:16])
    v = y[:, 16:80]
    zp = jnp.zeros((y.shape[0], 32), F32)
    t0_ref[...] = jnp.concatenate([q, q, k, k, v, zp], axis=1)
    d8 = 0.5 * _exp_precise(q * k)
    rep = (lax.broadcasted_iota(jnp.int32, (8, 64), 1) // DH
           == lax.broadcasted_iota(jnp.int32, (8, 64), 0)).astype(F32)
    ai = jnp.dot(d8, rep, preferred_element_type=F32, precision=HI) * v
    z16 = jnp.zeros((y.shape[0], 16), F32)
    ii_ref[...] = jnp.concatenate([d8, d8, z16, ai, zp], axis=1)


def _tc_b_body(ap_ref, b0_ref, wq_ref, bq_ref, wk_ref, bk_ref,
               wv_ref, t1_ref, i1_ref):
    d = ap_ref[0, :, 0:8] + ap_ref[1, :, 0:8]
    a = ap_ref[0, :, 32:96] + ap_ref[1, :, 32:96]
    inv = 1.0 / (d + 1e-16)
    rep = (lax.broadcasted_iota(jnp.int32, (8, 64), 1) // DH
           == lax.broadcasted_iota(jnp.int32, (8, 64), 0)).astype(F32)
    h0 = jax.nn.relu(
        a * jnp.dot(inv, rep, preferred_element_type=F32, precision=HI)
        + b0_ref[...])
    q1 = jax.nn.relu(jnp.dot(h0, wq_ref[...], preferred_element_type=F32,
                             precision=HI) + bq_ref[...])
    k1 = jax.nn.relu(jnp.dot(h0, wk_ref[...], preferred_element_type=F32,
                             precision=HI) + bk_ref[...])
    v1 = jnp.dot(h0, wv_ref[...], preferred_element_type=F32, precision=HI)
    rows = q1.shape[0]
    lanes16 = jnp.ones((1, 16), F32)
    ones = jnp.ones((rows, 1), F32)
    z8 = jnp.zeros((rows, 8), F32)
    z80 = jnp.zeros((rows, 80), F32)
    w1 = jnp.concatenate([ones, v1, z8], axis=1)
    t1_ref[...] = jnp.concatenate([q1 * lanes16, k1 * lanes16, w1, z80],
                                  axis=1)
    z32 = jnp.zeros((rows, 32), F32)
    i1_ref[...] = jnp.concatenate(
        [z32, (0.5 * _exp_precise(q1 * k1)) * w1, z80], axis=1)


def _tc_c_body(ap_ref, b1_ref, o_ref):
    s = ap_ref[0] + ap_ref[1]
    o_ref[...] = s[:, 33:40] / (s[:, 32:33] + 1e-16) + b1_ref[...]


def _take16(vec, pat):
    dn = lax.GatherDimensionNumbers(offset_dims=(), collapsed_slice_dims=(0,),
                                    start_index_map=(0,))
    return lax.gather(vec, pat[:, None], dn, slice_sizes=(1,),
                      mode=lax.GatherScatterMode.PROMISE_IN_BOUNDS)


def _sc_edge_pass(row_hbm, col_hbm, tab_hbm, init_hbm, aout_hbm,
                  rowt, colt, ga, gc, acc, sem_r, sem_c):
    c = lax.axis_index("c")
    s = lax.axis_index("s")
    wid = c * 16 + s
    rows = pl.ds(s * ROWS_PER_TILE, ROWS_PER_TILE)
    pltpu.sync_copy(init_hbm.at[rows], acc.at[rows])
    plsc.subcore_barrier()

    lane = lax.iota(jnp.int32, 16)
    pblk = EC_BLOCKS // 5

    @pl.loop(0, 5)
    def _phase(ph):
        off = pl.multiple_of(ph * pblk, 8)
        pltpu.sync_copy(row_hbm.at[wid, pl.ds(off, pblk)], rowt)
        pltpu.sync_copy(col_hbm.at[wid, pl.ds(off, pblk)], colt)

        @pl.loop(0, pblk)
        def _blk(b):
            ir = rowt.at[b]
            ic = colt.at[b]
            cp_r = pltpu.async_copy(tab_hbm.at[ir], ga, sem_r)
            cp_c = pltpu.async_copy(tab_hbm.at[ic], gc, sem_c)
            cp_r.wait()
            cp_c.wait()
            for e in range(EB):
                ex = jnp.exp(ga.at[e][pl.ds(0, 16)]
                             * gc.at[e][pl.ds(16, 16)])
                gc.at[e][pl.ds(0, 16)] = ex
                for r in range(4):
                    pat = (lane >= 8).astype(jnp.int32) + 2 * r
                    exf = _take16(ex, pat)
                    sl = pl.ds(32 + 16 * r, 16)
                    gc.at[e][sl] = exf * gc.at[e][sl]
            pltpu.sync_copy(gc, acc.at[ir], add=True)

    plsc.subcore_barrier()
    pltpu.sync_copy(acc.at[rows], aout_hbm.at[c, rows])


def kernel(x, edge_index, Wq0, bq0, Wk0, bk0, Wv0, b0, Wq1, bq1, Wk1, bk1,
           Wv1, b1):
    xp = jnp.pad(x, ((0, NP - N), (0, 0)))
    ei = edge_index.astype(jnp.int32)
    pad = jnp.full((EP - E,), N, jnp.int32)
    rowp = jnp.concatenate([ei[0], pad])
    colp = jnp.concatenate([ei[1], pad])
    row2d = rowp.reshape(32, EC_BLOCKS, EB)
    col2d = colp.reshape(32, EC_BLOCKS, EB)

    w0 = jnp.concatenate([Wq0, Wk0, Wv0], axis=1)
    b0cat = jnp.concatenate([bq0, bk0, jnp.zeros((64,), F32)]).reshape(1, 80)

    grid = NP // TCB
    t0, ii = pl.pallas_call(
        _tc_a_body,
        grid=(grid,),
        in_specs=[
            pl.BlockSpec((TCB, 128), lambda i: (i, 0)),
            pl.BlockSpec((128, 80), lambda i: (0, 0)),
            pl.BlockSpec((1, 80), lambda i: (0, 0)),
        ],
        out_specs=[
            pl.BlockSpec((TCB, 128), lambda i: (i, 0)),
            pl.BlockSpec((TCB, 128), lambda i: (i, 0)),
        ],
        out_shape=[
            jax.ShapeDtypeStruct((NP, 128), F32),
            jax.ShapeDtypeStruct((NP, 128), F32),
        ],
    )(xp, w0, b0cat)

    mesh = plsc.VectorSubcoreMesh(core_axis_name="c", subcore_axis_name="s")
    cp = pltpu.CompilerParams()
    if "needs_layout_passes" in pltpu.CompilerParams.__dataclass_fields__:
        cp = dataclasses.replace(cp, needs_layout_passes=False)
    edge_pass = functools.partial(
        pl.kernel,
        compiler_params=cp,
        out_type=jax.ShapeDtypeStruct((2, NP, 128), F32),
        mesh=mesh,
        scratch_types=[
            pltpu.VMEM((EC_BLOCKS // 5, EB), jnp.int32),
            pltpu.VMEM((EC_BLOCKS // 5, EB), jnp.int32),
            pltpu.VMEM((EB, 128), F32),
            pltpu.VMEM((EB, 128), F32),
            pltpu.VMEM_SHARED((NP, 128), F32),
            pltpu.SemaphoreType.DMA,
            pltpu.SemaphoreType.DMA,
        ],
    )(_sc_edge_pass)

    aparts = edge_pass(row2d, col2d, t0, ii)

    t1, i1 = pl.pallas_call(
        _tc_b_body,
        grid=(grid,),
        in_specs=[
            pl.BlockSpec((2, TCB, 128), lambda i: (0, i, 0)),
            pl.BlockSpec((1, 64), lambda i: (0, 0)),
            pl.BlockSpec((64, 1), lambda i: (0, 0)),
            pl.BlockSpec((1, 1), lambda i: (0, 0)),
            pl.BlockSpec((64, 1), lambda i: (0, 0)),
            pl.BlockSpec((1, 1), lambda i: (0, 0)),
            pl.BlockSpec((64, 7), lambda i: (0, 0)),
        ],
        out_specs=[
            pl.BlockSpec((TCB, 128), lambda i: (i, 0)),
            pl.BlockSpec((TCB, 128), lambda i: (i, 0)),
        ],
        out_shape=[
            jax.ShapeDtypeStruct((NP, 128), F32),
            jax.ShapeDtypeStruct((NP, 128), F32),
        ],
    )(aparts, b0.reshape(1, 64), Wq1, bq1.reshape(1, 1), Wk1,
      bk1.reshape(1, 1), Wv1)

    a1parts = edge_pass(row2d, col2d, t1, i1)

    out = pl.pallas_call(
        _tc_c_body,
        grid=(grid,),
        in_specs=[
            pl.BlockSpec((2, TCB, 128), lambda i: (0, i, 0)),
            pl.BlockSpec((1, 7), lambda i: (0, 0)),
        ],
        out_specs=pl.BlockSpec((TCB, 7), lambda i: (i, 0)),
        out_shape=jax.ShapeDtypeStruct((NP, 7), F32),
    )(a1parts, b1.reshape(1, 7))

    return out[:N]
